# Initial kernel scaffold; baseline (speedup 1.0000x reference)
import jax
import jax.numpy as jnp
from jax.experimental import pallas as pl


def kernel(x, edge_index, edge_attr, Wq, bq, Wk, bk, Wv, bv):
    raise NotImplementedError("write your pallas kernel here")


# speedup vs baseline: 32.2039x; 32.2039x over previous
import functools

import jax
import jax.numpy as jnp
import numpy as np
from jax import lax
from jax.experimental import pallas as pl
from jax.experimental.pallas import tpu as pltpu
from jax.experimental.pallas import tpu_sc as plsc

N = 10000
E = 320000
HID = 128
HEADS = 4
HEAD_DIM = 32
INV_SQRT_HD = 1.0 / float(np.sqrt(HEAD_DIM))

NCORES = 2
NSUB = 16
NW = NCORES * NSUB
EPW = E // NW
B = 40
NCH = EPW // B
STRIPE = N // NSUB
PIECE = 125
NPIECE = STRIPE // PIECE


def _qkv_body(x_ref, wq_ref, bq_ref, wk_ref, bk_ref, wv_ref, bv_ref,
              q_ref, k_ref, v_ref, skv_ref):
    x = x_ref[...]
    dn = (((1,), (1,)), ((), ()))
    q_ref[...] = lax.dot_general(x, wq_ref[...], dn,
                                 preferred_element_type=jnp.float32,
                                 precision=lax.Precision.HIGHEST) + bq_ref[...]
    k_ref[...] = lax.dot_general(x, wk_ref[...], dn,
                                 preferred_element_type=jnp.float32,
                                 precision=lax.Precision.HIGHEST) + bk_ref[...]
    v_ref[...] = lax.dot_general(x, wv_ref[...], dn,
                                 preferred_element_type=jnp.float32,
                                 precision=lax.Precision.HIGHEST) + bv_ref[...]
    sk = jnp.sum(wk_ref[...], axis=1)
    sv = jnp.sum(wv_ref[...], axis=1)
    skv_ref[...] = jnp.concatenate([sk[None, :], sv[None, :]], axis=0)


def _edge_body(q_hbm, k_hbm, v_hbm, skv_hbm, src_hbm, dst_hbm, ea_hbm,
               num_hbm, denx_hbm,
               src_v, dst_v, ea_v, qd_v, ks_v, vs_v, den_v, skv_v,
               denp_v, denx_v, sem1, sem2, sem3,
               num_sh, den_sh):
    c = lax.axis_index("c")
    s = lax.axis_index("s")
    wid = c * NSUB + s

    zero16 = jnp.zeros((16,), jnp.float32)

    def zrow128(r, _):
        for j in range(8):
            denx_v[r, pl.ds(16 * j, 16)] = zero16
        return 0
    lax.fori_loop(0, PIECE, zrow128, 0)

    def zrow16(r, _):
        denp_v[r, pl.ds(0, 16)] = zero16
        return 0
    lax.fori_loop(0, PIECE, zrow16, 0)

    row0 = s * STRIPE
    for p in range(NPIECE):
        pltpu.sync_copy(denx_v, num_sh.at[pl.ds(row0 + p * PIECE, PIECE)])
        pltpu.sync_copy(denp_v, den_sh.at[pl.ds(row0 + p * PIECE, PIECE)])
    plsc.subcore_barrier()

    pltpu.sync_copy(skv_hbm, skv_v)
    sk = [skv_v[0, pl.ds(16 * j, 16)] for j in range(8)]
    sv = [skv_v[1, pl.ds(16 * j, 16)] for j in range(8)]
    lane = lax.iota(jnp.int32, 16)
    lo8 = lane < 8

    def chunk_body(i, _):
        base = wid * EPW + i * B
        pltpu.sync_copy(src_hbm.at[pl.ds(base, B)], src_v)
        pltpu.sync_copy(dst_hbm.at[pl.ds(base, B)], dst_v)
        pltpu.sync_copy(ea_hbm.at[pl.ds(base, B)], ea_v.at[pl.ds(0, B)])
        cp1 = pltpu.async_copy(q_hbm.at[dst_v], qd_v, sem1)
        cp2 = pltpu.async_copy(k_hbm.at[src_v], ks_v, sem2)
        cp3 = pltpu.async_copy(v_hbm.at[src_v], vs_v, sem3)
        cp1.wait()
        cp2.wait()
        cp3.wait()

        def do_edge(e, ce):
            p = []
            for j in range(8):
                sl = pl.ds(16 * j, 16)
                kj = ks_v[e, sl] + ce * sk[j]
                p.append(qd_v[e, sl] * kj)
            a0 = jnp.sum(p[0] + p[1]) * INV_SQRT_HD
            a1 = jnp.sum(p[2] + p[3]) * INV_SQRT_HD
            a2 = jnp.sum(p[4] + p[5]) * INV_SQRT_HD
            a3 = jnp.sum(p[6] + p[7]) * INV_SQRT_HD
            t0 = jnp.exp(jnp.full((16,), a0, jnp.float32))
            t1 = jnp.exp(jnp.full((16,), a1, jnp.float32))
            t2 = jnp.exp(jnp.full((16,), a2, jnp.float32))
            t3 = jnp.exp(jnp.full((16,), a3, jnp.float32))
            den_v[e, pl.ds(0, 16)] = jnp.where(
                lane == 0, t0, jnp.where(
                    lane == 1, t1, jnp.where(
                        lane == 2, t2, jnp.where(
                            lane == 3, t3, zero16))))
            ts = (t0, t0, t1, t1, t2, t2, t3, t3)
            for j in range(8):
                sl = pl.ds(16 * j, 16)
                vj = vs_v[e, sl] + ce * sv[j]
                vs_v[e, sl] = ts[j] * vj

        def group_body(g, _):
            cv = ea_v[pl.ds(16 * g, 16)]
            for l in range(16):
                do_edge(16 * g + l, cv[l])
            return 0

        lax.fori_loop(0, B // 16, group_body, 0)
        if B % 16:
            cv = ea_v[pl.ds(B - B % 16, 16)]
            for l in range(B % 16):
                do_edge(B - B % 16 + l, cv[l])
        pltpu.sync_copy(vs_v, num_sh.at[dst_v], add=True)
        pltpu.sync_copy(den_v, den_sh.at[dst_v], add=True)
        return 0

    lax.fori_loop(0, NCH, chunk_body, 0)
    plsc.subcore_barrier()

    for pz in range(NPIECE):
        r0 = s * STRIPE + pz * PIECE
        pltpu.sync_copy(num_sh.at[pl.ds(r0, PIECE)],
                        num_hbm.at[c, pl.ds(r0, PIECE)])
        pltpu.sync_copy(den_sh.at[pl.ds(r0, PIECE)], denp_v)

        def xrow(r, _):
            d0 = denp_v[r, pl.ds(0, 16)]
            dvals = (d0[0], d0[1], d0[2], d0[3])
            for m in range(8):
                denx_v[r, pl.ds(16 * m, 16)] = jnp.full((16,), dvals[m // 2],
                                                        jnp.float32)
            return 0
        lax.fori_loop(0, PIECE, xrow, 0)
        pltpu.sync_copy(denx_v, denx_hbm.at[c, pl.ds(r0, PIECE)])


def _combine_body(num_ref, denx_ref, out_ref):
    n = num_ref[0] + num_ref[1]
    d = denx_ref[0] + denx_ref[1]
    out_ref[...] = n / (d + 1e-16)


@jax.jit
def kernel(x, edge_index, edge_attr, Wq, bq, Wk, bk, Wv, bv):
    ei = edge_index.astype(jnp.int32)
    src = ei[0]
    dst = ei[1]

    q, k, v, skv = pl.pallas_call(
        _qkv_body,
        out_shape=(
            jax.ShapeDtypeStruct((N, HID), jnp.float32),
            jax.ShapeDtypeStruct((N, HID), jnp.float32),
            jax.ShapeDtypeStruct((N, HID), jnp.float32),
            jax.ShapeDtypeStruct((2, HID), jnp.float32),
        ),
    )(x, Wq, bq.reshape(1, HID), Wk, bk.reshape(1, HID),
      Wv, bv.reshape(1, HID))

    mesh = plsc.VectorSubcoreMesh(core_axis_name="c", subcore_axis_name="s")
    edge_kernel = functools.partial(
        pl.kernel,
        out_type=(
            jax.ShapeDtypeStruct((NCORES, N, HID), jnp.float32),
            jax.ShapeDtypeStruct((NCORES, N, HID), jnp.float32),
        ),
        mesh=mesh,
        compiler_params=pltpu.CompilerParams(use_tc_tiling_on_sc=False,
                                             needs_layout_passes=False),
        scratch_types=[
            pltpu.VMEM((B,), jnp.int32),
            pltpu.VMEM((B,), jnp.int32),
            pltpu.VMEM((((B + 15) // 16) * 16,), jnp.float32),
            pltpu.VMEM((B, HID), jnp.float32),
            pltpu.VMEM((B, HID), jnp.float32),
            pltpu.VMEM((B, HID), jnp.float32),
            pltpu.VMEM((B, 16), jnp.float32),
            pltpu.VMEM((2, HID), jnp.float32),
            pltpu.VMEM((PIECE, 16), jnp.float32),
            pltpu.VMEM((PIECE, HID), jnp.float32),
            pltpu.SemaphoreType.DMA,
            pltpu.SemaphoreType.DMA,
            pltpu.SemaphoreType.DMA,
            pltpu.VMEM_SHARED((N, HID), jnp.float32),
            pltpu.VMEM_SHARED((N, 16), jnp.float32),
        ],
    )(_edge_body)
    num, denx = edge_kernel(q, k, v, skv, src, dst, edge_attr)

    RB = 2000
    out = pl.pallas_call(
        _combine_body,
        grid=(N // RB,),
        in_specs=[
            pl.BlockSpec((NCORES, RB, HID), lambda i: (0, i, 0)),
            pl.BlockSpec((NCORES, RB, HID), lambda i: (0, i, 0)),
        ],
        out_specs=pl.BlockSpec((RB, HID), lambda i: (i, 0)),
        out_shape=jax.ShapeDtypeStruct((N, HID), jnp.float32),
    )(num, denx)
    return out
